# Initial kernel scaffold; baseline (speedup 1.0000x reference)
import jax
import jax.numpy as jnp
from jax.experimental import pallas as pl


def kernel(y_real, y_imag, no, pilots_real, pilots_imag):
    raise NotImplementedError("write your pallas kernel here")


# speedup vs baseline: 1.4525x; 1.4525x over previous
import functools

import jax
import jax.numpy as jnp
from jax import lax
from jax.experimental import pallas as pl
from jax.experimental.pallas import tpu as pltpu
from jax.experimental.pallas import tpu_sc as plsc

B = 64
RX = 1
ANT = 16
T = 14
SC = 1024
P0, P1 = 2, 11

NC, NS, L = 2, 16, 16
NW = NC * NS
ROWS = (B * ANT) // NW
NCHUNK = SC // L
INV_DT = 1.0 / float(P1 - P0)


def _sc_body(yr_hbm, yi_hbm, pr_hbm, pi_hbm, out_hbm, pil_v, a_v, yin_v, out_v):
    cid = lax.axis_index("c")
    sid = lax.axis_index("s")
    wid = sid * NC + cid

    pltpu.sync_copy(pr_hbm.at[0], pil_v.at[0])
    pltpu.sync_copy(pi_hbm.at[0], pil_v.at[1])
    pltpu.sync_copy(pr_hbm.at[1], pil_v.at[2])
    pltpu.sync_copy(pi_hbm.at[1], pil_v.at[3])

    def factor_body(i, carry):
        s = pl.ds(i * L, L)
        for p in range(2):
            prv = pil_v[2 * p, s]
            piv = pil_v[2 * p + 1, s]
            inv = 1.0 / (prv * prv + piv * piv)
            a_v[2 * p, s] = prv * inv
            a_v[2 * p + 1, s] = piv * inv
        return carry

    lax.fori_loop(0, NCHUNK, factor_body, 0)

    def row_body(r, carry):
        row = wid * ROWS + r
        pltpu.sync_copy(yr_hbm.at[row, P0], yin_v.at[0])
        pltpu.sync_copy(yi_hbm.at[row, P0], yin_v.at[1])
        pltpu.sync_copy(yr_hbm.at[row, P1], yin_v.at[2])
        pltpu.sync_copy(yi_hbm.at[row, P1], yin_v.at[3])

        def chunk_body(i, c2):
            s = pl.ds(i * L, L)
            y0r = yin_v[0, s]
            y0i = yin_v[1, s]
            y1r = yin_v[2, s]
            y1i = yin_v[3, s]
            a0r = a_v[0, s]
            a0i = a_v[1, s]
            a1r = a_v[2, s]
            a1i = a_v[3, s]
            h0r = y0r * a0r + y0i * a0i
            h0i = y0i * a0r - y0r * a0i
            h1r = y1r * a1r + y1i * a1i
            h1i = y1i * a1r - y1r * a1i
            sr = (h1r - h0r) * INV_DT
            si = (h1i - h0i) * INV_DT
            for t in range(T):
                w = float(t - P0)
                out_v[0, t, s] = h0r + w * sr
                out_v[1, t, s] = h0i + w * si
            return c2

        lax.fori_loop(0, NCHUNK, chunk_body, 0)
        pltpu.sync_copy(out_v.at[0], out_hbm.at[0, row])
        pltpu.sync_copy(out_v.at[1], out_hbm.at[1, row])
        return carry

    lax.fori_loop(0, ROWS, row_body, 0)


@jax.jit
def _run(yr, yi, pr, pi):
    mesh = plsc.VectorSubcoreMesh(core_axis_name="c", subcore_axis_name="s",
                                  num_cores=NC, num_subcores=NS)
    k = functools.partial(
        pl.kernel,
        out_type=jax.ShapeDtypeStruct((2, B * ANT, T, SC), jnp.float32),
        mesh=mesh,
        scratch_types=[
            pltpu.VMEM((4, SC), jnp.float32),
            pltpu.VMEM((4, SC), jnp.float32),
            pltpu.VMEM((4, SC), jnp.float32),
            pltpu.VMEM((2, T, SC), jnp.float32),
        ],
    )(_sc_body)
    return k(yr, yi, pr, pi)


def kernel(y_real, y_imag, no, pilots_real, pilots_imag):
    yr = y_real.reshape(B * ANT, T, SC)
    yi = y_imag.reshape(B * ANT, T, SC)
    out = _run(yr, yi, pilots_real, pilots_imag)
    return out.reshape(2, B, RX, ANT, 1, 1, T, SC)


# speedup vs baseline: 1.8079x; 1.2447x over previous
import functools

import jax
import jax.numpy as jnp
from jax import lax
from jax.experimental import pallas as pl
from jax.experimental.pallas import tpu as pltpu
from jax.experimental.pallas import tpu_sc as plsc

B = 64
RX = 1
ANT = 16
T = 14
SC = 1024
P0, P1 = 2, 11

NC, NS, L = 2, 16, 16
NW = NC * NS
ROWS = (B * ANT) // NW
NCHUNK = SC // L
INV_DT = 1.0 / float(P1 - P0)


def _in_descs(yr_hbm, yi_hbm, yin_v, sem, b, row):
    return (
        (yr_hbm.at[row, P0], yin_v.at[b, 0], sem),
        (yi_hbm.at[row, P0], yin_v.at[b, 1], sem),
        (yr_hbm.at[row, P1], yin_v.at[b, 2], sem),
        (yi_hbm.at[row, P1], yin_v.at[b, 3], sem),
    )


def _out_descs(out_v, out_hbm, sem, b, row):
    return (
        (out_v.at[b, 0], out_hbm.at[0, row], sem),
        (out_v.at[b, 1], out_hbm.at[1, row], sem),
    )


def _sc_body(yr_hbm, yi_hbm, pr_hbm, pi_hbm, out_hbm,
             a_v, yin_v, out_v, s_in0, s_in1, s_out0, s_out1):
    cid = lax.axis_index("c")
    sid = lax.axis_index("s")
    wid = sid * NC + cid
    base = wid * ROWS
    s_in = (s_in0, s_in1)
    s_out = (s_out0, s_out1)

    pltpu.sync_copy(pr_hbm.at[0], yin_v.at[0, 0])
    pltpu.sync_copy(pi_hbm.at[0], yin_v.at[0, 1])
    pltpu.sync_copy(pr_hbm.at[1], yin_v.at[0, 2])
    pltpu.sync_copy(pi_hbm.at[1], yin_v.at[0, 3])

    def factor_body(i, carry):
        s = pl.ds(i * L, L)
        for p in range(2):
            prv = yin_v[0, 2 * p, s]
            piv = yin_v[0, 2 * p + 1, s]
            inv = 1.0 / (prv * prv + piv * piv)
            a_v[2 * p, s] = prv * inv
            a_v[2 * p + 1, s] = piv * inv
        return carry

    lax.fori_loop(0, NCHUNK, factor_body, 0)

    for d in _in_descs(yr_hbm, yi_hbm, yin_v, s_in[0], 0, base):
        pltpu.async_copy(*d)

    @pl.loop(0, ROWS, step=2)
    def row_loop(rr):
        for b in range(2):
            r = rr + b
            row = base + r
            for d in _in_descs(yr_hbm, yi_hbm, yin_v, s_in[b], b, row):
                pltpu.make_async_copy(*d).wait()
            @pl.when(r + 1 < ROWS)
            def _():
                for d in _in_descs(yr_hbm, yi_hbm, yin_v, s_in[1 - b],
                                   1 - b, row + 1):
                    pltpu.async_copy(*d)
            @pl.when(r >= 2)
            def _():
                for d in _out_descs(out_v, out_hbm, s_out[b], b, row):
                    pltpu.make_async_copy(*d).wait()

            @plsc.parallel_loop(0, NCHUNK, unroll=4)
            def chunk_body(i):
                s = pl.ds(i * L, L)
                y0r = yin_v[b, 0, s]
                y0i = yin_v[b, 1, s]
                y1r = yin_v[b, 2, s]
                y1i = yin_v[b, 3, s]
                a0r = a_v[0, s]
                a0i = a_v[1, s]
                a1r = a_v[2, s]
                a1i = a_v[3, s]
                h0r = y0r * a0r + y0i * a0i
                h0i = y0i * a0r - y0r * a0i
                h1r = y1r * a1r + y1i * a1i
                h1i = y1i * a1r - y1r * a1i
                sr = (h1r - h0r) * INV_DT
                si = (h1i - h0i) * INV_DT
                for t in range(T):
                    w = float(t - P0)
                    out_v[b, 0, t, s] = h0r + w * sr
                    out_v[b, 1, t, s] = h0i + w * si

            for d in _out_descs(out_v, out_hbm, s_out[b], b, row):
                pltpu.async_copy(*d)

    for b in range(2):
        for d in _out_descs(out_v, out_hbm, s_out[b], b, base + ROWS - 2 + b):
            pltpu.make_async_copy(*d).wait()


@jax.jit
def _run(yr, yi, pr, pi):
    mesh = plsc.VectorSubcoreMesh(core_axis_name="c", subcore_axis_name="s",
                                  num_cores=NC, num_subcores=NS)
    k = functools.partial(
        pl.kernel,
        out_type=jax.ShapeDtypeStruct((2, B * ANT, T, SC), jnp.float32),
        mesh=mesh,
        scratch_types=[
            pltpu.VMEM((4, SC), jnp.float32),
            pltpu.VMEM((2, 4, SC), jnp.float32),
            pltpu.VMEM((2, 2, T, SC), jnp.float32),
            pltpu.SemaphoreType.DMA,
            pltpu.SemaphoreType.DMA,
            pltpu.SemaphoreType.DMA,
            pltpu.SemaphoreType.DMA,
        ],
    )(_sc_body)
    return k(yr, yi, pr, pi)


def kernel(y_real, y_imag, no, pilots_real, pilots_imag):
    yr = y_real.reshape(B * ANT, T, SC)
    yi = y_imag.reshape(B * ANT, T, SC)
    out = _run(yr, yi, pilots_real, pilots_imag)
    return out.reshape(2, B, RX, ANT, 1, 1, T, SC)
